# baseline (speedup 1.0000x reference)
import functools

import jax
import jax.numpy as jnp
from jax import lax
from jax.experimental import pallas as pl
from jax.experimental.pallas import tpu as pltpu
from jax.experimental.pallas import tpu_sc as plsc

NC = 2
NS = 16
NW = NC * NS

S = 16384
T = 50
D = 64
B = S * T
R = B // 2
R_PER_W = R // NW
CHUNK = 160
N_CHUNKS = R_PER_W // CHUNK
NBUF = 4
MAIN_G = (N_CHUNKS - NBUF) // NBUF
assert R_PER_W % CHUNK == 0 and N_CHUNKS % NBUF == 0 and CHUNK % 8 == 0

_mesh = plsc.VectorSubcoreMesh(core_axis_name="c", subcore_axis_name="s")


@functools.partial(
    pl.kernel,
    out_type=jax.ShapeDtypeStruct((R, 2 * D), jnp.float32),
    mesh=_mesh,
    scratch_types=(
        [pltpu.VMEM((R_PER_W,), jnp.int32) for _ in range(2)]
        + [pltpu.VMEM((CHUNK, D), jnp.float32) for _ in range(2 * NBUF)]
        + [pltpu.SemaphoreType.DMA for _ in range(4 * NBUF)]
    ),
    compiler_params=pltpu.CompilerParams(use_tc_tiling_on_sc=False),
)
def _gather_kernel(ev_hbm, od_hbm, table_hbm, out_hbm, ev_v, od_v, *scratch):
    bufs_l = scratch[:NBUF]
    bufs_r = scratch[NBUF:2 * NBUF]
    gsems_a = scratch[2 * NBUF:3 * NBUF]
    gsems_b = scratch[3 * NBUF:4 * NBUF]
    osems_a = scratch[4 * NBUF:5 * NBUF]
    osems_b = scratch[5 * NBUF:]

    wid = lax.axis_index("s") * NC + lax.axis_index("c")
    wbase = wid * R_PER_W

    def cs(i):
        return pl.ds(pl.multiple_of(i * CHUNK, 8), CHUNK)

    def os_(i):
        return pl.ds(pl.multiple_of(wbase + i * CHUNK, 8), CHUNK)

    hs = pl.ds(pl.multiple_of(wbase, 8), R_PER_W)
    pltpu.sync_copy(ev_hbm.at[hs], ev_v)
    pltpu.sync_copy(od_hbm.at[hs], od_v)

    def fire_gathers(b, i):
        pltpu.async_copy(table_hbm.at[ev_v.at[cs(i)]], bufs_l[b], gsems_a[b])
        pltpu.async_copy(table_hbm.at[od_v.at[cs(i)]], bufs_r[b], gsems_b[b])

    def wait_gathers(b, i):
        pltpu.make_async_copy(table_hbm.at[ev_v.at[cs(i)]], bufs_l[b],
                              gsems_a[b]).wait()
        pltpu.make_async_copy(table_hbm.at[od_v.at[cs(i)]], bufs_r[b],
                              gsems_b[b]).wait()

    def fire_out(b, i):
        pltpu.async_copy(bufs_l[b], out_hbm.at[os_(i), pl.ds(0, D)],
                         osems_a[b])
        pltpu.async_copy(bufs_r[b], out_hbm.at[os_(i), pl.ds(D, D)],
                         osems_b[b])

    def wait_out(b, i):
        pltpu.make_async_copy(bufs_l[b], out_hbm.at[os_(i), pl.ds(0, D)],
                              osems_a[b]).wait()
        pltpu.make_async_copy(bufs_r[b], out_hbm.at[os_(i), pl.ds(D, D)],
                              osems_b[b]).wait()

    for b in range(NBUF):
        fire_gathers(b, b)

    @pl.loop(0, MAIN_G)
    def main(g):
        for b in range(NBUF):
            i = g * NBUF + b
            wait_gathers(b, i)
            fire_out(b, i)
            wait_out(b, i)
            fire_gathers(b, i + NBUF)

    for b in range(NBUF):
        i = MAIN_G * NBUF + b
        wait_gathers(b, i)
        fire_out(b, i)
    for b in range(NBUF):
        i = MAIN_G * NBUF + b
        wait_out(b, i)


def kernel(token_ids, weight):
    flat = token_ids.reshape(-1).astype(jnp.int32)
    pairs = flat.reshape(R, 2)
    out2 = _gather_kernel(pairs[:, 0], pairs[:, 1], weight)
    return out2.reshape(S, T, D)
